# baseline (speedup 1.0000x reference)
import functools

import jax
import jax.numpy as jnp
from jax import lax
from jax.experimental import pallas as pl
from jax.experimental.pallas import tpu as pltpu
from jax.experimental.pallas import tpu_sc as plsc

_NC = 2
_NS = 16
_NW = _NC * _NS
_D = 128
_NV = _D // 16
_EPS = 1e-12
_CH = 40


def _make_fused(n_rows: int, T: int):
    rows_per_w = n_rows // _NW
    n_chunks = rows_per_w // _CH
    n_pairs = n_chunks // 2
    mesh = plsc.VectorSubcoreMesh(core_axis_name="c", subcore_axis_name="s")

    @functools.partial(
        pl.kernel,
        out_type=jax.ShapeDtypeStruct((n_rows, _D), jnp.float32),
        mesh=mesh,
        scratch_types=[
            pltpu.VMEM((rows_per_w,), jnp.int32),
            pltpu.VMEM((_CH, _D), jnp.float32),
            pltpu.VMEM((_CH, _D), jnp.float32),
            pltpu.VMEM((_CH, _D), jnp.float32),
            pltpu.VMEM((_CH, _D), jnp.float32),
            pltpu.VMEM((T, _D), jnp.float32),
            pltpu.VMEM((2, _D), jnp.float32),
            pltpu.VMEM((_D,), jnp.float32),
            pltpu.VMEM((_D,), jnp.float32),
            pltpu.SemaphoreType.DMA,
            pltpu.SemaphoreType.DMA,
            pltpu.SemaphoreType.DMA,
            pltpu.SemaphoreType.DMA,
        ],
    )
    def fused_kernel(ids_hbm, table_hbm, pos_hbm, tt_hbm, gamma_hbm, beta_hbm,
                     out_hbm, idx_v, buf0, buf1, obuf0, obuf1, bias_v, tt_v,
                     gamma_v, beta_v, gs0, gs1, ss0, ss1):
        wid = lax.axis_index("s") * _NC + lax.axis_index("c")
        base = wid * rows_per_w
        pltpu.sync_copy(ids_hbm.at[pl.ds(base, rows_per_w)], idx_v)
        pltpu.sync_copy(pos_hbm.at[pl.ds(0, T)], bias_v)
        pltpu.sync_copy(tt_hbm, tt_v)
        pltpu.sync_copy(gamma_hbm, gamma_v)
        pltpu.sync_copy(beta_hbm, beta_v)

        tt_row = [tt_v[0, pl.ds(16 * j, 16)] for j in range(_NV)]

        def bias_body(r):
            for j in range(_NV):
                sl = pl.ds(16 * j, 16)
                bias_v[r, sl] = bias_v[r, sl] + tt_row[j]

        plsc.parallel_loop(0, T, 1, unroll=4)(bias_body)

        g_vec = [gamma_v[pl.ds(16 * j, 16)] for j in range(_NV)]
        b_vec = [beta_v[pl.ds(16 * j, 16)] for j in range(_NV)]

        dnums = lax.GatherDimensionNumbers(
            offset_dims=(), collapsed_slice_dims=(0,), start_index_map=(0,))
        iota16 = lax.iota(jnp.int32, 16)
        perm_idx = [jnp.bitwise_xor(iota16, sh).reshape(16, 1)
                    for sh in (8, 4, 2, 1)]

        def _lane_bcast_sum(v):
            for pidx in perm_idx:
                v = v + lax.gather(
                    v, pidx, dnums, slice_sizes=(1,),
                    mode=lax.GatherScatterMode.PROMISE_IN_BOUNDS)
            return v

        def compute(buf, obuf, t0):
            def row_body(r):
                h = [buf[r, pl.ds(16 * j, 16)] + bias_v[t0 + r, pl.ds(16 * j, 16)]
                     for j in range(_NV)]
                s = h[0]
                ss = h[0] * h[0]
                for j in range(1, _NV):
                    s = s + h[j]
                    ss = ss + h[j] * h[j]
                mean = _lane_bcast_sum(s) * (1.0 / _D)
                tot2 = _lane_bcast_sum(ss) * (1.0 / _D)
                x = tot2 - mean * mean + _EPS
                xi = lax.bitcast_convert_type(x, jnp.int32)
                yi = 0x5F3759DF - lax.shift_right_logical(xi, 1)
                y = lax.bitcast_convert_type(yi, jnp.float32)
                hx = x * 0.5
                y = y * (1.5 - hx * y * y)
                y = y * (1.5 - hx * y * y)
                y = y * (1.5 - hx * y * y)
                for j in range(_NV):
                    obuf[r, pl.ds(16 * j, 16)] = (
                        (h[j] - mean) * (y * g_vec[j]) + b_vec[j])

            plsc.parallel_loop(0, _CH, 1, unroll=8)(row_body)

        def gather_start(g, buf, sem):
            off = lax.rem(g, n_chunks) * _CH
            pltpu.async_copy(table_hbm.at[idx_v.at[pl.ds(off, _CH)]], buf, sem)

        def gather_wait(buf, sem):
            pltpu.make_async_copy(
                table_hbm.at[idx_v.at[pl.ds(0, _CH)]], buf, sem).wait()

        def scatter_start(g, obuf, sem):
            off = g * _CH
            pltpu.async_copy(obuf, out_hbm.at[pl.ds(base + off, _CH)], sem)

        def scatter_drain(obuf, sem):
            pltpu.make_async_copy(
                obuf, out_hbm.at[pl.ds(base, _CH)], sem).wait()

        gather_start(0, buf0, gs0)

        def pair_body(p, c):
            g0 = 2 * p
            g1 = g0 + 1
            gather_start(g1, buf1, gs1)
            gather_wait(buf0, gs0)

            @pl.when(p > 0)
            def _():
                scatter_drain(obuf0, ss0)

            compute(buf0, obuf0, lax.rem(g0 * _CH, T))
            scatter_start(g0, obuf0, ss0)
            gather_start(g0 + 2, buf0, gs0)
            gather_wait(buf1, gs1)

            @pl.when(p > 0)
            def _():
                scatter_drain(obuf1, ss1)

            compute(buf1, obuf1, lax.rem(g1 * _CH, T))
            scatter_start(g1, obuf1, ss1)
            return c

        lax.fori_loop(0, n_pairs, pair_body, 0)
        gather_wait(buf0, gs0)
        scatter_drain(obuf0, ss0)
        scatter_drain(obuf1, ss1)

    return fused_kernel


def kernel(input_ids, W_word, W_pos, W_tt, gamma, beta):
    B, T = input_ids.shape
    ids_flat = input_ids.reshape(-1).astype(jnp.int32)
    out = _make_fused(B * T, T)(ids_flat, W_word, W_pos, W_tt, gamma, beta)
    return out.reshape(B, T, _D)


# speedup vs baseline: 1.6991x; 1.6991x over previous
import functools

import jax
import jax.numpy as jnp
from jax import lax
from jax.experimental import pallas as pl
from jax.experimental.pallas import tpu as pltpu
from jax.experimental.pallas import tpu_sc as plsc

_NC = 2
_NS = 16
_NW = _NC * _NS
_D = 128
_CHUNK = 200


def _make_sc_gather(n_rows: int):
    rows_per_w = n_rows // _NW
    n_chunks = rows_per_w // _CHUNK
    n_quads = n_chunks // 4
    mesh = plsc.VectorSubcoreMesh(core_axis_name="c", subcore_axis_name="s")

    @functools.partial(
        pl.kernel,
        out_type=jax.ShapeDtypeStruct((n_rows, _D), jnp.float32),
        mesh=mesh,
        scratch_types=[
            pltpu.VMEM((rows_per_w,), jnp.int32),
            pltpu.VMEM((_CHUNK, _D), jnp.float32),
            pltpu.VMEM((_CHUNK, _D), jnp.float32),
            pltpu.VMEM((_CHUNK, _D), jnp.float32),
            pltpu.VMEM((_CHUNK, _D), jnp.float32),
            pltpu.SemaphoreType.DMA,
            pltpu.SemaphoreType.DMA,
            pltpu.SemaphoreType.DMA,
            pltpu.SemaphoreType.DMA,
            pltpu.SemaphoreType.DMA,
            pltpu.SemaphoreType.DMA,
            pltpu.SemaphoreType.DMA,
            pltpu.SemaphoreType.DMA,
        ],
    )
    def gather_kernel(ids_hbm, table_hbm, out_hbm, idx_v, b0, b1, b2, b3,
                      gs0, gs1, gs2, gs3, ws0, ws1, ws2, ws3):
        wid = lax.axis_index("s") * _NC + lax.axis_index("c")
        base = wid * rows_per_w
        pltpu.sync_copy(ids_hbm.at[pl.ds(base, rows_per_w)], idx_v)

        bufs = (b0, b1, b2, b3)
        gsems = (gs0, gs1, gs2, gs3)
        wsems = (ws0, ws1, ws2, ws3)

        def gather_start(g, j):
            pltpu.async_copy(
                table_hbm.at[idx_v.at[pl.ds(g * _CHUNK, _CHUNK)]],
                bufs[j], gsems[j])

        def gather_wait(j):
            pltpu.make_async_copy(
                table_hbm.at[idx_v.at[pl.ds(0, _CHUNK)]], bufs[j],
                gsems[j]).wait()

        def write_start(g, j):
            pltpu.async_copy(
                bufs[j], out_hbm.at[pl.ds(base + g * _CHUNK, _CHUNK)],
                wsems[j])

        def write_wait(j):
            pltpu.make_async_copy(
                bufs[j], out_hbm.at[pl.ds(base, _CHUNK)], wsems[j]).wait()

        gather_start(0, 0)
        gather_start(1, 1)

        def quad_body(q, carry):
            for j in range(4):
                g = 4 * q + j
                jn = (j + 2) % 4

                @pl.when(g >= 2)
                def _():
                    write_wait(jn)

                @pl.when(g + 2 < n_chunks)
                def _():
                    gather_start(g + 2, jn)

                gather_wait(j)
                write_start(g, j)
            return carry

        lax.fori_loop(0, n_quads, quad_body, 0)
        write_wait((n_chunks - 2) % 4)
        write_wait((n_chunks - 1) % 4)

    return gather_kernel


_ROWS_BLK = 1600
_EPS = 1e-12


def _ln_body(x_ref, pos_ref, tt_ref, gamma_ref, beta_ref, o_ref):
    x = x_ref[...].reshape(_ROWS_BLK // 200, 200, _D)
    bias = pos_ref[...] + tt_ref[0][None, :]
    h = x + bias[None]
    mean = jnp.mean(h, axis=-1, keepdims=True)
    c = h - mean
    var = jnp.mean(c * c, axis=-1, keepdims=True)
    normed = c * lax.rsqrt(var + _EPS)
    out = normed * gamma_ref[0][None, None, :] + beta_ref[0][None, None, :]
    o_ref[...] = out.reshape(_ROWS_BLK, _D)


def _layernorm(gathered, W_pos_t, W_tt, gamma2d, beta2d):
    n_rows = gathered.shape[0]
    grid = (n_rows // _ROWS_BLK,)
    return pl.pallas_call(
        _ln_body,
        grid=grid,
        in_specs=[
            pl.BlockSpec((_ROWS_BLK, _D), lambda i: (i, 0)),
            pl.BlockSpec((200, _D), lambda i: (0, 0)),
            pl.BlockSpec((2, _D), lambda i: (0, 0)),
            pl.BlockSpec((1, _D), lambda i: (0, 0)),
            pl.BlockSpec((1, _D), lambda i: (0, 0)),
        ],
        out_specs=pl.BlockSpec((_ROWS_BLK, _D), lambda i: (i, 0)),
        out_shape=jax.ShapeDtypeStruct((n_rows, _D), jnp.float32),
    )(gathered, W_pos_t, W_tt, gamma2d, beta2d)


def kernel(input_ids, W_word, W_pos, W_tt, gamma, beta):
    B, T = input_ids.shape
    ids_flat = input_ids.reshape(-1).astype(jnp.int32)
    gathered = _make_sc_gather(B * T)(ids_flat, W_word)
    out = _layernorm(
        gathered,
        W_pos[:T],
        W_tt,
        gamma.reshape(1, _D),
        beta.reshape(1, _D),
    )
    return out.reshape(B, T, _D)


# speedup vs baseline: 2.1879x; 1.2876x over previous
import functools

import jax
import jax.numpy as jnp
from jax import lax
from jax.experimental import pallas as pl
from jax.experimental.pallas import tpu as pltpu
from jax.experimental.pallas import tpu_sc as plsc

_NC = 2
_NS = 16
_NW = _NC * _NS
_D = 128
_CHUNK = 200


def _make_sc_gather(n_rows: int):
    rows_per_w = n_rows // _NW
    n_chunks = rows_per_w // _CHUNK
    n_quads = n_chunks // 4
    mesh = plsc.VectorSubcoreMesh(core_axis_name="c", subcore_axis_name="s")

    @functools.partial(
        pl.kernel,
        out_type=jax.ShapeDtypeStruct((n_rows, _D), jnp.float32),
        mesh=mesh,
        scratch_types=[
            pltpu.VMEM((rows_per_w,), jnp.int32),
            pltpu.VMEM((_CHUNK, _D), jnp.float32),
            pltpu.VMEM((_CHUNK, _D), jnp.float32),
            pltpu.VMEM((_CHUNK, _D), jnp.float32),
            pltpu.VMEM((_CHUNK, _D), jnp.float32),
            pltpu.SemaphoreType.DMA,
            pltpu.SemaphoreType.DMA,
            pltpu.SemaphoreType.DMA,
            pltpu.SemaphoreType.DMA,
            pltpu.SemaphoreType.DMA,
            pltpu.SemaphoreType.DMA,
            pltpu.SemaphoreType.DMA,
            pltpu.SemaphoreType.DMA,
        ],
    )
    def gather_kernel(ids_hbm, table_hbm, out_hbm, idx_v, b0, b1, b2, b3,
                      gs0, gs1, gs2, gs3, ws0, ws1, ws2, ws3):
        wid = lax.axis_index("s") * _NC + lax.axis_index("c")
        base = wid * rows_per_w
        pltpu.sync_copy(ids_hbm.at[pl.ds(base, rows_per_w)], idx_v)

        bufs = (b0, b1, b2, b3)
        gsems = (gs0, gs1, gs2, gs3)
        wsems = (ws0, ws1, ws2, ws3)

        def gather_start(g, j):
            pltpu.async_copy(
                table_hbm.at[idx_v.at[pl.ds(g * _CHUNK, _CHUNK)]],
                bufs[j], gsems[j])

        def gather_wait(j):
            pltpu.make_async_copy(
                table_hbm.at[idx_v.at[pl.ds(0, _CHUNK)]], bufs[j],
                gsems[j]).wait()

        def write_start(g, j):
            pltpu.async_copy(
                bufs[j], out_hbm.at[pl.ds(base + g * _CHUNK, _CHUNK)],
                wsems[j])

        def write_wait(j):
            pltpu.make_async_copy(
                bufs[j], out_hbm.at[pl.ds(base, _CHUNK)], wsems[j]).wait()

        gather_start(0, 0)
        gather_start(1, 1)

        def quad_body(q, carry):
            for j in range(4):
                g = 4 * q + j
                jn = (j + 2) % 4

                @pl.when(g >= 2)
                def _():
                    write_wait(jn)

                @pl.when(g + 2 < n_chunks)
                def _():
                    gather_start(g + 2, jn)

                gather_wait(j)
                write_start(g, j)
            return carry

        lax.fori_loop(0, n_quads, quad_body, 0)
        write_wait((n_chunks - 2) % 4)
        write_wait((n_chunks - 1) % 4)

    return gather_kernel


_ROWS_BLK = 6400
_EPS = 1e-12


def _ln_body(x_ref, pos_ref, tt_ref, gamma_ref, beta_ref, o_ref):
    x = x_ref[...].reshape(_ROWS_BLK // 200, 200, _D)
    bias = pos_ref[...] + tt_ref[0][None, :]
    h = x + bias[None]
    mean = jnp.mean(h, axis=-1, keepdims=True)
    c = h - mean
    var = jnp.mean(c * c, axis=-1, keepdims=True)
    normed = c * lax.rsqrt(var + _EPS)
    out = normed * gamma_ref[0][None, None, :] + beta_ref[0][None, None, :]
    o_ref[...] = out.reshape(_ROWS_BLK, _D)


def _layernorm(gathered, W_pos_t, W_tt, gamma2d, beta2d):
    n_rows = gathered.shape[0]
    grid = (n_rows // _ROWS_BLK,)
    return pl.pallas_call(
        _ln_body,
        grid=grid,
        in_specs=[
            pl.BlockSpec((_ROWS_BLK, _D), lambda i: (i, 0)),
            pl.BlockSpec((200, _D), lambda i: (0, 0)),
            pl.BlockSpec((2, _D), lambda i: (0, 0)),
            pl.BlockSpec((1, _D), lambda i: (0, 0)),
            pl.BlockSpec((1, _D), lambda i: (0, 0)),
        ],
        out_specs=pl.BlockSpec((_ROWS_BLK, _D), lambda i: (i, 0)),
        out_shape=jax.ShapeDtypeStruct((n_rows, _D), jnp.float32),
    )(gathered, W_pos_t, W_tt, gamma2d, beta2d)


def kernel(input_ids, W_word, W_pos, W_tt, gamma, beta):
    B, T = input_ids.shape
    ids_flat = input_ids.reshape(-1).astype(jnp.int32)
    gathered = _make_sc_gather(B * T)(ids_flat, W_word)
    out = _layernorm(
        gathered,
        W_pos[:T],
        W_tt,
        gamma.reshape(1, _D),
        beta.reshape(1, _D),
    )
    return out.reshape(B, T, _D)


# speedup vs baseline: 2.2528x; 1.0297x over previous
import functools

import jax
import jax.numpy as jnp
from jax import lax
from jax.experimental import pallas as pl
from jax.experimental.pallas import tpu as pltpu
from jax.experimental.pallas import tpu_sc as plsc

_NC = 2
_NS = 16
_NW = _NC * _NS
_D = 128
_CHUNK = 200


def _make_sc_gather(n_rows: int):
    rows_per_w = n_rows // _NW
    n_chunks = rows_per_w // _CHUNK
    n_quads = n_chunks // 4
    mesh = plsc.VectorSubcoreMesh(core_axis_name="c", subcore_axis_name="s")

    @functools.partial(
        pl.kernel,
        out_type=jax.ShapeDtypeStruct((n_rows, _D), jnp.float32),
        mesh=mesh,
        scratch_types=[
            pltpu.VMEM((rows_per_w,), jnp.int32),
            pltpu.VMEM((_CHUNK, _D), jnp.float32),
            pltpu.VMEM((_CHUNK, _D), jnp.float32),
            pltpu.VMEM((_CHUNK, _D), jnp.float32),
            pltpu.VMEM((_CHUNK, _D), jnp.float32),
            pltpu.SemaphoreType.DMA,
            pltpu.SemaphoreType.DMA,
            pltpu.SemaphoreType.DMA,
            pltpu.SemaphoreType.DMA,
            pltpu.SemaphoreType.DMA,
            pltpu.SemaphoreType.DMA,
            pltpu.SemaphoreType.DMA,
            pltpu.SemaphoreType.DMA,
        ],
    )
    def gather_kernel(ids_hbm, table_hbm, out_hbm, idx_v, b0, b1, b2, b3,
                      gs0, gs1, gs2, gs3, ws0, ws1, ws2, ws3):
        wid = lax.axis_index("s") * _NC + lax.axis_index("c")
        base = wid * rows_per_w
        pltpu.sync_copy(ids_hbm.at[pl.ds(base, rows_per_w)], idx_v)

        bufs = (b0, b1, b2, b3)
        gsems = (gs0, gs1, gs2, gs3)
        wsems = (ws0, ws1, ws2, ws3)

        def gather_start(g, j):
            pltpu.async_copy(
                table_hbm.at[idx_v.at[pl.ds(g * _CHUNK, _CHUNK)]],
                bufs[j], gsems[j])

        def gather_wait(j):
            pltpu.make_async_copy(
                table_hbm.at[idx_v.at[pl.ds(0, _CHUNK)]], bufs[j],
                gsems[j]).wait()

        def write_start(g, j):
            pltpu.async_copy(
                bufs[j], out_hbm.at[pl.ds(base + g * _CHUNK, _CHUNK)],
                wsems[j])

        def write_wait(j):
            pltpu.make_async_copy(
                bufs[j], out_hbm.at[pl.ds(base, _CHUNK)], wsems[j]).wait()

        gather_start(0, 0)
        gather_start(1, 1)

        def quad_body(q, carry):
            for j in range(4):
                g = 4 * q + j
                jn = (j + 2) % 4

                @pl.when(g >= 2)
                def _():
                    write_wait(jn)

                @pl.when(g + 2 < n_chunks)
                def _():
                    gather_start(g + 2, jn)

                gather_wait(j)
                write_start(g, j)
            return carry

        lax.fori_loop(0, n_quads, quad_body, 0)
        write_wait((n_chunks - 2) % 4)
        write_wait((n_chunks - 1) % 4)

    return gather_kernel


_ROWS_BLK = 12800
_EPS = 1e-12


def _ln_body(x_ref, pos_ref, tt_ref, gamma_ref, beta_ref, o_ref):
    x = x_ref[...].reshape(_ROWS_BLK // 200, 200, _D)
    bias = pos_ref[...] + tt_ref[0][None, :]
    h = x + bias[None]
    mean = jnp.mean(h, axis=-1, keepdims=True)
    c = h - mean
    var = jnp.mean(c * c, axis=-1, keepdims=True)
    normed = c * lax.rsqrt(var + _EPS)
    out = normed * gamma_ref[0][None, None, :] + beta_ref[0][None, None, :]
    o_ref[...] = out.reshape(_ROWS_BLK, _D)


def _layernorm(gathered, W_pos_t, W_tt, gamma2d, beta2d):
    n_rows = gathered.shape[0]
    grid = (n_rows // _ROWS_BLK,)
    return pl.pallas_call(
        _ln_body,
        grid=grid,
        in_specs=[
            pl.BlockSpec((_ROWS_BLK, _D), lambda i: (i, 0)),
            pl.BlockSpec((200, _D), lambda i: (0, 0)),
            pl.BlockSpec((2, _D), lambda i: (0, 0)),
            pl.BlockSpec((1, _D), lambda i: (0, 0)),
            pl.BlockSpec((1, _D), lambda i: (0, 0)),
        ],
        out_specs=pl.BlockSpec((_ROWS_BLK, _D), lambda i: (i, 0)),
        out_shape=jax.ShapeDtypeStruct((n_rows, _D), jnp.float32),
    )(gathered, W_pos_t, W_tt, gamma2d, beta2d)


def kernel(input_ids, W_word, W_pos, W_tt, gamma, beta):
    B, T = input_ids.shape
    ids_flat = input_ids.reshape(-1).astype(jnp.int32)
    gathered = _make_sc_gather(B * T)(ids_flat, W_word)
    out = _layernorm(
        gathered,
        W_pos[:T],
        W_tt,
        gamma.reshape(1, _D),
        beta.reshape(1, _D),
    )
    return out.reshape(B, T, _D)
